# baseline (speedup 1.0000x reference)
import functools
import math

import jax
import jax.numpy as jnp
from jax import lax
from jax.experimental import pallas as pl
from jax.experimental.pallas import tpu as pltpu
from jax.experimental.pallas import tpu_sc as plsc

_MARGIN = 0


def _sc_gather(labels, indexes):
    b = indexes.shape[0]
    info = plsc.get_sparse_core_info()
    num_cores = info.num_cores
    nw = info.num_cores * info.num_subcores
    bpw = b // nw

    mesh = plsc.VectorSubcoreMesh(core_axis_name="c", subcore_axis_name="s")

    @functools.partial(
        pl.kernel,
        mesh=mesh,
        out_type=jax.ShapeDtypeStruct((b,), jnp.int32),
        scratch_types=[
            pltpu.VMEM((bpw,), jnp.int32),
            pltpu.VMEM((bpw,), jnp.int32),
            pltpu.SemaphoreType.DMA,
        ],
    )
    def sc_kernel(labels_hbm, idx_hbm, y_hbm, idx_v, y_v, sem):
        wid = lax.axis_index("s") * num_cores + lax.axis_index("c")
        base = wid * bpw
        pltpu.sync_copy(idx_hbm.at[pl.ds(base, bpw)], idx_v)
        pltpu.async_copy(labels_hbm.at[idx_v], y_v, sem).wait()
        pltpu.sync_copy(y_v, y_hbm.at[pl.ds(base, bpw)])

    return sc_kernel(labels, indexes)


def _tc_body(n_cols, n_blocks, blk_r, batch,
             x_ref, lab_ref, y_ref, idx_ref,
             loss_ref, min_ref, mean_ref, p_acc, z_acc):
    j = pl.program_id(0)
    ch = 256
    n_full = n_cols // ch
    tail = n_cols - n_full * ch

    y = y_ref[...]
    idx = idx_ref[...]
    iot = lax.broadcasted_iota(jnp.int32, (1, ch), 1)

    def chunk_sums(base, width, iota_w):
        xc = x_ref[:, pl.ds(base, width)]
        lc = lab_ref[:, pl.ds(base, width)]
        e = jnp.exp(xc)
        e = jnp.where(base + iota_w != idx, e, 0.0)
        return e, jnp.where(lc == y, e, 0.0)

    def step(c, accs):
        az, ap = accs
        e, s = chunk_sums(c * ch, ch, iot)
        return az + e, ap + s

    az, ap = lax.fori_loop(
        0, n_full, step,
        (jnp.zeros((blk_r, ch), jnp.float32),
         jnp.zeros((blk_r, ch), jnp.float32)))
    zsum = jnp.sum(az, axis=1, keepdims=True)
    psum = jnp.sum(ap, axis=1, keepdims=True)
    if tail:
        iot_t = lax.broadcasted_iota(jnp.int32, (1, tail), 1)
        e, s = chunk_sums(n_full * ch, tail, iot_t)
        zsum = zsum + jnp.sum(e, axis=1, keepdims=True)
        psum = psum + jnp.sum(s, axis=1, keepdims=True)
    rows = pl.ds(j * blk_r, blk_r)
    z_acc[rows, :] = zsum
    p_acc[rows, :] = psum

    @pl.when(j == n_blocks - 1)
    def _fin():
        p = p_acc[...] * (1.0 / math.exp(_MARGIN))
        z = (z_acc[...] - p_acc[...]) + p
        prob = p / z
        nzm = prob != 0.0
        logp = jnp.where(nzm, jnp.log(jnp.where(nzm, prob, 1.0)), 0.0)
        loss_ref[...] = jnp.full((1, 1), -1.0 / batch) * jnp.sum(logp)
        min_ref[...] = jnp.full((1, 1), 1.0) * jnp.min(p)
        mean_ref[...] = jnp.full((1, 1), 1.0 / batch) * jnp.sum(p)


def _tc_reduce(x, labels2d, y2d, idx2d, blk_r=32):
    batch, n_cols = x.shape
    n_blocks = batch // blk_r
    out11 = jax.ShapeDtypeStruct((1, 1), jnp.float32)
    body = functools.partial(_tc_body, n_cols, n_blocks, blk_r, batch)
    return pl.pallas_call(
        body,
        grid=(n_blocks,),
        in_specs=[
            pl.BlockSpec((blk_r, n_cols), lambda j: (j, 0)),
            pl.BlockSpec((1, n_cols), lambda j: (0, 0)),
            pl.BlockSpec((blk_r, 1), lambda j: (j, 0)),
            pl.BlockSpec((blk_r, 1), lambda j: (j, 0)),
        ],
        out_specs=[
            pl.BlockSpec((1, 1), lambda j: (0, 0)),
            pl.BlockSpec((1, 1), lambda j: (0, 0)),
            pl.BlockSpec((1, 1), lambda j: (0, 0)),
        ],
        out_shape=[out11, out11, out11],
        scratch_shapes=[
            pltpu.VMEM((batch, 1), jnp.float32),
            pltpu.VMEM((batch, 1), jnp.float32),
        ],
        compiler_params=pltpu.CompilerParams(
            dimension_semantics=("arbitrary",),
        ),
    )(x, labels2d, y2d, idx2d)


def kernel(x, features, labels, indexes):
    del features
    batch, n_cols = x.shape
    y = _sc_gather(labels, indexes)
    loss, pmin, pmean = _tc_reduce(
        x,
        labels.reshape(1, n_cols),
        y.reshape(batch, 1),
        indexes.reshape(batch, 1),
    )
    return (loss[0, 0], pmin[0, 0], pmean[0, 0])


# speedup vs baseline: 2.6192x; 2.6192x over previous
import functools
import math

import jax
import jax.numpy as jnp
from jax import lax
from jax.experimental import pallas as pl
from jax.experimental.pallas import tpu as pltpu
from jax.experimental.pallas import tpu_sc as plsc

_MARGIN = 0


def _sc_gather(labels, indexes):
    b = indexes.shape[0]
    info = plsc.get_sparse_core_info()
    num_cores = info.num_cores
    nw = info.num_cores * info.num_subcores
    bpw = b // nw

    mesh = plsc.VectorSubcoreMesh(core_axis_name="c", subcore_axis_name="s")

    @functools.partial(
        pl.kernel,
        mesh=mesh,
        out_type=jax.ShapeDtypeStruct((b,), jnp.int32),
        scratch_types=[
            pltpu.VMEM((bpw,), jnp.int32),
            pltpu.VMEM((bpw,), jnp.int32),
            pltpu.SemaphoreType.DMA,
        ],
    )
    def sc_kernel(labels_hbm, idx_hbm, y_hbm, idx_v, y_v, sem):
        wid = lax.axis_index("s") * num_cores + lax.axis_index("c")
        base = wid * bpw
        pltpu.sync_copy(idx_hbm.at[pl.ds(base, bpw)], idx_v)
        pltpu.async_copy(labels_hbm.at[idx_v], y_v, sem).wait()
        pltpu.sync_copy(y_v, y_hbm.at[pl.ds(base, bpw)])

    return sc_kernel(labels, indexes)


def _tc_body(n_cols, n_blocks, blk_w, batch,
             xa_ref, xb_ref, lab_ref, y_ref, idx_ref,
             loss_ref, min_ref, mean_ref, p_acc, z_acc):
    j = pl.program_id(0)

    @pl.when(j == 0)
    def _init():
        p_acc[...] = jnp.zeros_like(p_acc)
        z_acc[...] = jnp.zeros_like(z_acc)

    half = batch // 2
    col = j * blk_w + lax.broadcasted_iota(jnp.int32, (1, blk_w), 1)
    valid = col < n_cols
    lab = lab_ref[...]
    for x_ref, r0 in ((xa_ref, 0), (xb_ref, half)):
        rows = pl.ds(r0, half)
        e = jnp.exp(x_ref[...])
        e = jnp.where(valid & (col != idx_ref[rows, :]), e, 0.0)
        same = lab == y_ref[rows, :]
        z_acc[rows, :] += jnp.sum(e, axis=1, keepdims=True)
        p_acc[rows, :] += jnp.sum(jnp.where(same, e, 0.0), axis=1,
                                  keepdims=True)

    @pl.when(j == n_blocks - 1)
    def _fin():
        p = p_acc[...] * (1.0 / math.exp(_MARGIN))
        z = (z_acc[...] - p_acc[...]) + p
        prob = p / z
        nzm = prob != 0.0
        logp = jnp.where(nzm, jnp.log(jnp.where(nzm, prob, 1.0)), 0.0)
        loss_ref[...] = jnp.full((1, 1), -1.0 / batch) * jnp.sum(logp)
        min_ref[...] = jnp.full((1, 1), 1.0) * jnp.min(p)
        mean_ref[...] = jnp.full((1, 1), 1.0 / batch) * jnp.sum(p)


def _tc_reduce(x, labels2d, y2d, idx2d, blk_w=4096):
    batch, n_cols = x.shape
    half = batch // 2
    n_blocks = pl.cdiv(n_cols, blk_w)
    out11 = jax.ShapeDtypeStruct((1, 1), jnp.float32)
    body = functools.partial(_tc_body, n_cols, n_blocks, blk_w, batch)
    return pl.pallas_call(
        body,
        grid=(n_blocks,),
        in_specs=[
            pl.BlockSpec((half, blk_w), lambda j: (0, j)),
            pl.BlockSpec((half, blk_w), lambda j: (1, j)),
            pl.BlockSpec((1, blk_w), lambda j: (0, j)),
            pl.BlockSpec((batch, 1), lambda j: (0, 0)),
            pl.BlockSpec((batch, 1), lambda j: (0, 0)),
        ],
        out_specs=[
            pl.BlockSpec((1, 1), lambda j: (0, 0)),
            pl.BlockSpec((1, 1), lambda j: (0, 0)),
            pl.BlockSpec((1, 1), lambda j: (0, 0)),
        ],
        out_shape=[out11, out11, out11],
        scratch_shapes=[
            pltpu.VMEM((batch, 1), jnp.float32),
            pltpu.VMEM((batch, 1), jnp.float32),
        ],
        compiler_params=pltpu.CompilerParams(
            dimension_semantics=("arbitrary",),
        ),
    )(x, x, labels2d, y2d, idx2d)


def kernel(x, features, labels, indexes):
    del features
    batch, n_cols = x.shape
    y = _sc_gather(labels, indexes)
    loss, pmin, pmean = _tc_reduce(
        x,
        labels.reshape(1, n_cols),
        y.reshape(batch, 1),
        indexes.reshape(batch, 1),
    )
    return (loss[0, 0], pmin[0, 0], pmean[0, 0])
